# baseline (speedup 1.0000x reference)
import functools

import jax
import jax.numpy as jnp
import numpy as np
from jax import lax
from jax.experimental import pallas as pl
from jax.experimental.pallas import tpu as pltpu
from jax.experimental.pallas import tpu_sc as plsc

NUM_SENSE = 3
EMB_DIM = 64
ROW3 = NUM_SENSE * EMB_DIM
LANES = 16
NK = EMB_DIM // LANES

NC = 2
NS = 16
NW = NC * NS
EB = 4

_TINY = float(np.finfo(np.float32).tiny)


def _pad8(x):
    return (x + 7) // 8 * 8


def _vdot(ref, row, s, v4):
    acc = ref[row, pl.ds(s * EMB_DIM, LANES)] * v4[0]
    for k in range(1, NK):
        acc = acc + ref[row, pl.ds(s * EMB_DIM + k * LANES, LANES)] * v4[k]
    return jnp.sum(acc)


def _row4(ref, row, s):
    return [ref[row, pl.ds(s * EMB_DIM + k * LANES, LANES)] for k in range(NK)]


def _store_scalar_vec(ref, idx, splat_vec):
    lane = lax.iota(jnp.int32, LANES)
    plsc.store_scatter(ref, [jnp.full((LANES,), idx, jnp.int32)],
                       splat_vec, mask=lane == 0)


def _softmax3_vec(t0, t1, t2):
    m = jnp.maximum(jnp.maximum(t0, t1), t2)
    e0 = jnp.exp(jnp.broadcast_to(t0 - m, (LANES,)))
    e1 = jnp.exp(jnp.broadcast_to(t1 - m, (LANES,)))
    e2 = jnp.exp(jnp.broadcast_to(t2 - m, (LANES,)))
    inv = 1.0 / (e0 + e1 + e2)
    return e0 * inv, e1 * inv, e2 * inv


def _sc_body(emb3, dis3, glob, icg, iw, ing,
             p_out, q_out,
             v_icg, v_iw, v_ing,
             bufs_a, bufs_b, pbuf, qbuf, sems_a, sems_b,
             *, c, n, n_eb):
    wid = lax.axis_index("s") * NC + lax.axis_index("c")
    ecb = EB * c
    enb = EB * n
    enb_p = _pad8(enb)
    ewb_p = _pad8(EB)
    wctx = n_eb * ecb
    wneg = n_eb * enb

    pltpu.sync_copy(icg.at[pl.ds(wid * wctx, wctx)], v_icg)
    pltpu.sync_copy(iw.at[pl.ds(wid * (n_eb * ewb_p), n_eb * ewb_p)], v_iw)
    pltpu.sync_copy(ing.at[pl.ds(wid * (n_eb * enb_p), n_eb * enb_p)], v_ing)

    def streams(i, bufs, sems):
        be, bd, bg, bng, bwe, bwd = bufs
        gg = v_icg.at[pl.ds(i * ecb, ecb)]
        nn = v_ing.at[pl.ds(i * enb_p, enb_p)]
        ww = v_iw.at[pl.ds(i * ewb_p, ewb_p)]
        return ((dis3.at[gg], bd, sems[0]), (emb3.at[gg], be, sems[1]),
                (glob.at[gg], bg, sems[2]), (glob.at[nn], bng, sems[3]),
                (emb3.at[ww], bwe, sems[4]), (dis3.at[ww], bwd, sems[5]))

    def issue(i, bufs, sems):
        for src, dst, sem in streams(i, bufs, sems):
            pltpu.async_copy(src, dst, sem)

    def wait(i, bufs, sems):
        for src, dst, sem in streams(i, bufs, sems):
            pltpu.make_async_copy(src, dst, sem).wait()

    def compute(i, bufs):
        be, bd, bg, bng, bwe, bwd = bufs
        for b in range(EB):
            base_c = b * c

            def cacc(cc, acc4):
                r4 = _row4(bg, base_c + cc, 0)
                return [acc4[k] + r4[k] for k in range(NK)]
            v0 = lax.fori_loop(0, c, cacc, [jnp.zeros((LANES,), jnp.float32)] * NK)
            v0 = [x * (1.0 / c) for x in v0]

            def dis_pass(v4):
                def body(cc, u4):
                    row = base_c + cc
                    a0, a1, a2 = _softmax3_vec(
                        _vdot(bd, row, 0, v4), _vdot(bd, row, 1, v4),
                        _vdot(bd, row, 2, v4))
                    r0 = _row4(be, row, 0)
                    r1 = _row4(be, row, 1)
                    r2 = _row4(be, row, 2)
                    return [u4[k] + a0 * r0[k] + a1 * r1[k] + a2 * r2[k]
                            for k in range(NK)]
                u4 = lax.fori_loop(0, c, body, [jnp.zeros((LANES,), jnp.float32)] * NK)
                return [x * (1.0 / c) for x in u4]

            v3 = dis_pass(dis_pass(v0))

            aw0, aw1, aw2 = _softmax3_vec(
                _vdot(bwd, b, 0, v3), _vdot(bwd, b, 1, v3), _vdot(bwd, b, 2, v3))
            we0 = _row4(bwe, b, 0)
            we1 = _row4(bwe, b, 1)
            we2 = _row4(bwe, b, 2)

            def mix_prob(rows_ref, row):
                t0 = _vdot(rows_ref, row, 0, we0)
                t1 = _vdot(rows_ref, row, 0, we1)
                t2 = _vdot(rows_ref, row, 0, we2)
                g0 = 1.0 / (1.0 + jnp.exp(jnp.broadcast_to(-t0, (LANES,))))
                g1 = 1.0 / (1.0 + jnp.exp(jnp.broadcast_to(-t1, (LANES,))))
                g2 = 1.0 / (1.0 + jnp.exp(jnp.broadcast_to(-t2, (LANES,))))
                return aw0 * g0 + aw1 * g1 + aw2 * g2

            def pos_body(cc, carry):
                _store_scalar_vec(pbuf, i * ecb + base_c + cc,
                                  mix_prob(bg, base_c + cc))
                return carry

            lax.fori_loop(0, c, pos_body, 0)

            def neg_body(nb, carry):
                _store_scalar_vec(qbuf, i * enb + b * n + nb,
                                  mix_prob(bng, b * n + nb))
                return carry

            lax.fori_loop(0, n, neg_body, 0)

    issue(0, bufs_a, sems_a)

    def pair_step(i2, carry):
        c0 = 2 * i2
        issue(c0 + 1, bufs_b, sems_b)
        wait(c0, bufs_a, sems_a)
        compute(c0, bufs_a)

        @pl.when(c0 + 2 < n_eb)
        def _():
            issue(c0 + 2, bufs_a, sems_a)

        wait(c0 + 1, bufs_b, sems_b)
        compute(c0 + 1, bufs_b)
        return carry

    lax.fori_loop(0, n_eb // 2, pair_step, 0)

    pltpu.sync_copy(pbuf, p_out.at[pl.ds(wid * wctx, wctx)])
    pltpu.sync_copy(qbuf, q_out.at[pl.ds(wid * wneg, wneg)])


def _sc_probs(emb_w, dis_w, glob, word_ids, context_ids, neg_ids):
    b, c = context_ids.shape
    n = neg_ids.shape[1]
    vocab = glob.shape[0]
    wb = b // NW
    n_eb = wb // EB
    ecb, enb = EB * c, EB * n
    enb_p = _pad8(enb)
    ewb_p = _pad8(EB)

    emb3 = emb_w.reshape(vocab, ROW3)
    dis3 = dis_w.reshape(vocab, ROW3)
    icg = context_ids.reshape(-1)
    iw = jnp.concatenate(
        [word_ids.reshape(b // EB, EB),
         jnp.zeros((b // EB, ewb_p - EB), word_ids.dtype)], axis=1).reshape(-1)
    ing = jnp.concatenate(
        [neg_ids.reshape(b // EB, enb),
         jnp.zeros((b // EB, enb_p - enb), neg_ids.dtype)], axis=1).reshape(-1)

    f32 = jnp.float32
    mesh = plsc.VectorSubcoreMesh(core_axis_name="c", subcore_axis_name="s",
                                  num_cores=NC, num_subcores=NS)

    def bufset():
        return [pltpu.VMEM((ecb, ROW3), f32),
                pltpu.VMEM((ecb, ROW3), f32),
                pltpu.VMEM((ecb, EMB_DIM), f32),
                pltpu.VMEM((enb_p, EMB_DIM), f32),
                pltpu.VMEM((ewb_p, ROW3), f32),
                pltpu.VMEM((ewb_p, ROW3), f32)]

    run = pl.kernel(
        functools.partial(_sc_body, c=c, n=n, n_eb=n_eb),
        out_type=[jax.ShapeDtypeStruct((b * c,), f32),
                  jax.ShapeDtypeStruct((b * n,), f32)],
        mesh=mesh,
        compiler_params=pltpu.CompilerParams(use_tc_tiling_on_sc=False,
                                             needs_layout_passes=False),
        scratch_types=(
            [pltpu.VMEM((wb * c,), jnp.int32),
             pltpu.VMEM((n_eb * ewb_p,), jnp.int32),
             pltpu.VMEM((n_eb * enb_p,), jnp.int32)]
            + [bufset(), bufset()]
            + [pltpu.VMEM((wb * c,), f32),
               pltpu.VMEM((wb * n,), f32)]
            + [[pltpu.SemaphoreType.DMA] * 6, [pltpu.SemaphoreType.DMA] * 6]
        ),
    )
    return run(emb3, dis3, glob, icg, iw, ing)


def _loss_body(p_ref, q_ref, out_ref, *, denom):
    p = p_ref[...]
    q = q_ref[...]
    loss = (-jnp.sum(jnp.log(jnp.maximum(p, _TINY)))
            - jnp.sum(jnp.log(jnp.maximum(1.0 - q, _TINY))))
    out_ref[0, 0] = loss * denom


def _loss(p, q, b, c, n):
    out = pl.pallas_call(
        functools.partial(_loss_body, denom=1.0 / (b * c)),
        in_specs=[pl.BlockSpec((b, c), lambda: (0, 0)),
                  pl.BlockSpec((b, n), lambda: (0, 0))],
        out_specs=pl.BlockSpec((1, 1), lambda: (0, 0), memory_space=pltpu.SMEM),
        out_shape=jax.ShapeDtypeStruct((1, 1), jnp.float32),
    )(p.reshape(b, c), q.reshape(b, n))
    return out[0, 0]


def kernel(word_ids, context_ids, context_masks, neg_ids, emb_w, global_w, disamb_w):
    del context_masks
    b, c = context_ids.shape
    n = neg_ids.shape[1]
    p, q = _sc_probs(emb_w, disamb_w, global_w, word_ids, context_ids, neg_ids)
    return _loss(p, q, b, c, n)
